# baseline (speedup 1.0000x reference)
import jax
import jax.numpy as jnp
from jax.experimental import pallas as pl

N = 10000
H = 8
D = 64


def _combine_body(agg_ref, skip_ref, o_ref):
    o_ref[...] = agg_ref[...] + skip_ref[...]


def kernel(x, edge_index, Wq, bq, Wk, bk, Wv, bv, Wskip, bskip):
    src = edge_index[0]
    dst = edge_index[1]
    q = (x @ Wq + bq).reshape(N, H, D)
    k = (x @ Wk + bk).reshape(N, H, D)
    v = (x @ Wv + bv).reshape(N, H, D)
    alpha = jnp.sum(q[dst] * k[src], axis=-1) / jnp.sqrt(jnp.float32(D))
    m = jax.ops.segment_max(alpha, dst, num_segments=N)
    m = jnp.where(jnp.isfinite(m), m, 0.0)
    ex = jnp.exp(alpha - m[dst])
    s = jax.ops.segment_sum(ex, dst, num_segments=N)
    attn = ex / (s[dst] + 1e-16)
    agg = jax.ops.segment_sum(attn[:, :, None] * v[src], dst, num_segments=N)
    agg = agg.mean(axis=1)
    skip = x @ Wskip + bskip
    out = pl.pallas_call(
        _combine_body,
        out_shape=jax.ShapeDtypeStruct((N, D), jnp.float32),
    )(agg, skip)
    return out


# speedup vs baseline: 1.1119x; 1.1119x over previous
import functools

import jax
import jax.numpy as jnp
from jax import lax
from jax.experimental import pallas as pl
from jax.experimental.pallas import tpu as pltpu
from jax.experimental.pallas import tpu_sc as plsc

N = 10000
E = 160000
F = 256
H = 8
D = 64
HD = H * D
BN = 1000
SCALE = 0.125

NC = 2
NS = 16
L = 16
NW = NC * NS
EW = E // NW
WA = 64
NWIN = (EW + WA - 1) // WA
EP = E + WA

_MESH = plsc.VectorSubcoreMesh(core_axis_name="c", subcore_axis_name="s",
                               num_cores=NC, num_subcores=NS)


def _worker_id():
    return lax.axis_index("s") * NC + lax.axis_index("c")


def _alpha_body(q_hbm, k_hbm, src_hbm, dst_hbm, alpha_hbm,
                sidx, didx, qd, ks, at, semq, semk):
    wid = _worker_id()
    wbase = wid * EW

    def win(j, _):
        base = wbase + j * WA
        pltpu.sync_copy(src_hbm.at[pl.ds(base, WA)], sidx)
        pltpu.sync_copy(dst_hbm.at[pl.ds(base, WA)], didx)
        cq = pltpu.async_copy(q_hbm.at[didx], qd, semq)
        ck = pltpu.async_copy(k_hbm.at[sidx], ks, semk)
        cq.wait()
        ck.wait()

        lane = lax.iota(jnp.int32, L)

        def group(g, _):
            def edge(j, accs):
                e = g * L + j
                new = []
                for h in range(H):
                    o = h * D
                    p = qd[e, pl.ds(o, L)] * ks[e, pl.ds(o, L)]
                    p = p + qd[e, pl.ds(o + L, L)] * ks[e, pl.ds(o + L, L)]
                    p = (p + qd[e, pl.ds(o + 2 * L, L)] *
                         ks[e, pl.ds(o + 2 * L, L)])
                    p = (p + qd[e, pl.ds(o + 3 * L, L)] *
                         ks[e, pl.ds(o + 3 * L, L)])
                    s = jnp.sum(p) * SCALE
                    new.append(jnp.where(lane == j, s, accs[h]))
                return tuple(new)

            accs = lax.fori_loop(0, L, edge,
                                 tuple(jnp.zeros((L,), jnp.float32)
                                       for _ in range(H)))
            for h in range(H):
                at[h, pl.ds(g * L, L)] = accs[h]
            return 0

        lax.fori_loop(0, WA // L, group, 0)
        for h in range(H):
            pltpu.sync_copy(at.at[h], alpha_hbm.at[pl.ds(h * EP + base, WA)])
        return 0

    lax.fori_loop(0, NWIN, win, 0)


@functools.partial(
    pl.kernel,
    out_type=jax.ShapeDtypeStruct((H * EP,), jnp.float32),
    mesh=_MESH,
    compiler_params=pltpu.CompilerParams(needs_layout_passes=False),
    scratch_types=[
        pltpu.VMEM((WA,), jnp.int32),
        pltpu.VMEM((WA,), jnp.int32),
        pltpu.VMEM((WA, HD), jnp.float32),
        pltpu.VMEM((WA, HD), jnp.float32),
        pltpu.VMEM((H, WA), jnp.float32),
        pltpu.SemaphoreType.DMA,
        pltpu.SemaphoreType.DMA,
    ],
)
def _sc_alpha(q_hbm, k_hbm, src_hbm, dst_hbm, alpha_hbm,
              sidx, didx, qd, ks, at, semq, semk):
    _alpha_body(q_hbm, k_hbm, src_hbm, dst_hbm, alpha_hbm,
                sidx, didx, qd, ks, at, semq, semk)


def _proj_body(x_ref, wq_ref, bq_ref, wk_ref, bk_ref, wv_ref, bv_ref,
               q_ref, k_ref, vt_ref):
    xb = x_ref[...]
    q_ref[...] = jnp.dot(xb, wq_ref[...],
                         preferred_element_type=jnp.float32) + bq_ref[...]
    k_ref[...] = jnp.dot(xb, wk_ref[...],
                         preferred_element_type=jnp.float32) + bk_ref[...]
    bv = bv_ref[...]
    vt_ref[0, ...] = jnp.dot(xb, wv_ref[:, :D],
                             preferred_element_type=jnp.float32) + bv[:, :D]
    vt_ref[1, ...] = jnp.dot(xb, wv_ref[:, D:],
                             preferred_element_type=jnp.float32) + bv[:, D:]


def _projections(x, Wq, bq, Wk, bk, Wv, bv):
    grid = (N // BN, H // 2)
    w_spec = pl.BlockSpec((F, 2 * D), lambda i, h: (0, h))
    b_spec = pl.BlockSpec((1, 2 * D), lambda i, h: (0, h))
    return pl.pallas_call(
        _proj_body,
        grid=grid,
        in_specs=[
            pl.BlockSpec((BN, F), lambda i, h: (i, 0)),
            w_spec, b_spec, w_spec, b_spec, w_spec, b_spec,
        ],
        out_specs=[
            pl.BlockSpec((BN, 2 * D), lambda i, h: (i, h)),
            pl.BlockSpec((BN, 2 * D), lambda i, h: (i, h)),
            pl.BlockSpec((2, BN, D), lambda i, h: (h, i, 0)),
        ],
        out_shape=[
            jax.ShapeDtypeStruct((N, HD), jnp.float32),
            jax.ShapeDtypeStruct((N, HD), jnp.float32),
            jax.ShapeDtypeStruct((H, N, D), jnp.float32),
        ],
    )(x, Wq, bq.reshape(1, HD), Wk, bk.reshape(1, HD), Wv, bv.reshape(1, HD))


def _combine_body(p0_ref, p1_ref, x_ref, ws_ref, bs_ref, o_ref):
    acc = jnp.sum(p0_ref[...], axis=0) + jnp.sum(p1_ref[...], axis=0)
    o_ref[...] = acc * jnp.float32(1.0 / H) + jnp.dot(
        x_ref[...], ws_ref[...], preferred_element_type=jnp.float32) + bs_ref[...]


def _combine(p0, p1, x, Wskip, bskip):
    grid = (N // BN,)
    return pl.pallas_call(
        _combine_body,
        grid=grid,
        in_specs=[
            pl.BlockSpec((H, BN, D), lambda i: (0, i, 0)),
            pl.BlockSpec((H, BN, D), lambda i: (0, i, 0)),
            pl.BlockSpec((BN, F), lambda i: (i, 0)),
            pl.BlockSpec((F, D), lambda i: (0, 0)),
            pl.BlockSpec((1, D), lambda i: (0, 0)),
        ],
        out_specs=pl.BlockSpec((BN, D), lambda i: (i, 0)),
        out_shape=jax.ShapeDtypeStruct((N, D), jnp.float32),
    )(p0, p1, x, Wskip, bskip.reshape(1, D))


def kernel(x, edge_index, Wq, bq, Wk, bk, Wv, bv, Wskip, bskip):
    src = edge_index[0]
    dst = edge_index[1]
    srcp = jnp.concatenate([src, jnp.zeros((EP - E,), jnp.int32)])
    dstp = jnp.concatenate([dst, jnp.zeros((EP - E,), jnp.int32)])
    q, k, vt = _projections(x, Wq, bq, Wk, bk, Wv, bv)

    alpha_he = _sc_alpha(q, k, srcp, dstp).reshape(H, EP)

    vh = vt
    alpha = alpha_he[:, :E].T
    ex = jnp.exp(alpha)
    s = jax.ops.segment_sum(ex, dst, num_segments=N)
    attn = ex / (s[dst] + 1e-16)
    agg = jax.vmap(lambda a, v_: jax.ops.segment_sum(a[:, None] * v_, dst,
                                                     num_segments=N))(attn.T,
                                                                      vh[:, src, :])
    p0 = agg
    p1 = jnp.zeros_like(agg)
    return _combine(p0, p1, x, Wskip, bskip)


# speedup vs baseline: 13.2597x; 11.9257x over previous
import functools

import jax
import jax.numpy as jnp
from jax import lax
from jax.experimental import pallas as pl
from jax.experimental.pallas import tpu as pltpu
from jax.experimental.pallas import tpu_sc as plsc

N = 10000
E = 160000
F = 256
H = 8
D = 64
HD = H * D
BN = 1000
SCALE = 0.125

NC = 2
NS = 16
L = 16
NW = NC * NS
EW = E // NW
WA = 64
NWIN = (EW + WA - 1) // WA
EP = E + WA

_MESH = plsc.VectorSubcoreMesh(core_axis_name="c", subcore_axis_name="s",
                               num_cores=NC, num_subcores=NS)


def _worker_id():
    return lax.axis_index("s") * NC + lax.axis_index("c")


def _alpha_body(q_hbm, k_hbm, src_hbm, dst_hbm, alpha_hbm,
                sidx, didx, qd, ks, at, semq, semk):
    wid = _worker_id()
    wbase = wid * EW

    def win(j, _):
        base = wbase + j * WA
        pltpu.sync_copy(src_hbm.at[pl.ds(base, WA)], sidx)
        pltpu.sync_copy(dst_hbm.at[pl.ds(base, WA)], didx)
        cq = pltpu.async_copy(q_hbm.at[didx], qd, semq)
        ck = pltpu.async_copy(k_hbm.at[sidx], ks, semk)
        cq.wait()
        ck.wait()

        lane = lax.iota(jnp.int32, L)

        def group(g, _):
            def edge(j, accs):
                e = g * L + j
                new = []
                for h in range(H):
                    o = h * D
                    p = qd[e, pl.ds(o, L)] * ks[e, pl.ds(o, L)]
                    p = p + qd[e, pl.ds(o + L, L)] * ks[e, pl.ds(o + L, L)]
                    p = (p + qd[e, pl.ds(o + 2 * L, L)] *
                         ks[e, pl.ds(o + 2 * L, L)])
                    p = (p + qd[e, pl.ds(o + 3 * L, L)] *
                         ks[e, pl.ds(o + 3 * L, L)])
                    s = jnp.sum(p) * SCALE
                    new.append(jnp.where(lane == j, s, accs[h]))
                return tuple(new)

            accs = lax.fori_loop(0, L, edge,
                                 tuple(jnp.zeros((L,), jnp.float32)
                                       for _ in range(H)))
            for h in range(H):
                at[h, pl.ds(g * L, L)] = accs[h]
            return 0

        lax.fori_loop(0, WA // L, group, 0)
        for h in range(H):
            pltpu.sync_copy(at.at[h], alpha_hbm.at[pl.ds(h * EP + base, WA)])
        return 0

    lax.fori_loop(0, NWIN, win, 0)


@functools.partial(
    pl.kernel,
    out_type=jax.ShapeDtypeStruct((H * EP,), jnp.float32),
    mesh=_MESH,
    compiler_params=pltpu.CompilerParams(needs_layout_passes=False),
    scratch_types=[
        pltpu.VMEM((WA,), jnp.int32),
        pltpu.VMEM((WA,), jnp.int32),
        pltpu.VMEM((WA, HD), jnp.float32),
        pltpu.VMEM((WA, HD), jnp.float32),
        pltpu.VMEM((H, WA), jnp.float32),
        pltpu.SemaphoreType.DMA,
        pltpu.SemaphoreType.DMA,
    ],
)
def _sc_alpha(q_hbm, k_hbm, src_hbm, dst_hbm, alpha_hbm,
              sidx, didx, qd, ks, at, semq, semk):
    _alpha_body(q_hbm, k_hbm, src_hbm, dst_hbm, alpha_hbm,
                sidx, didx, qd, ks, at, semq, semk)


def _proj_body(x_ref, wq_ref, bq_ref, wk_ref, bk_ref, wv_ref, bv_ref,
               q_ref, k_ref, vt_ref):
    xb = x_ref[...]
    q_ref[...] = jnp.dot(xb, wq_ref[...],
                         preferred_element_type=jnp.float32) + bq_ref[...]
    k_ref[...] = jnp.dot(xb, wk_ref[...],
                         preferred_element_type=jnp.float32) + bk_ref[...]
    vt_ref[...] = jnp.dot(xb, wv_ref[...],
                          preferred_element_type=jnp.float32) + bv_ref[...]


def _projections(x, Wq, bq, Wk, bk, Wv, bv):
    grid = (N // BN, H // 2)
    w_spec = pl.BlockSpec((F, 2 * D), lambda i, h: (0, h))
    b_spec = pl.BlockSpec((1, 2 * D), lambda i, h: (0, h))
    return pl.pallas_call(
        _proj_body,
        grid=grid,
        in_specs=[
            pl.BlockSpec((BN, F), lambda i, h: (i, 0)),
            w_spec, b_spec, w_spec, b_spec, w_spec, b_spec,
        ],
        out_specs=[
            pl.BlockSpec((BN, 2 * D), lambda i, h: (i, h)),
            pl.BlockSpec((BN, 2 * D), lambda i, h: (i, h)),
            pl.BlockSpec((BN, 2 * D), lambda i, h: (h * (N // BN) + i, 0)),
        ],
        out_shape=[
            jax.ShapeDtypeStruct((N, HD), jnp.float32),
            jax.ShapeDtypeStruct((N, HD), jnp.float32),
            jax.ShapeDtypeStruct((4 * N, 2 * D), jnp.float32),
        ],
    )(x, Wq, bq.reshape(1, HD), Wk, bk.reshape(1, HD), Wv, bv.reshape(1, HD))


NHP = 81920
WB = 1000
NGB = WB // L + 1


def _seg_body(alpha_hbm, dst_hbm, out_hbm, a_w, dst_w, vals, sidx, zbuf, s_sp):
    cid = lax.axis_index("c")
    sid = lax.axis_index("s")
    wid = sid * NC + cid
    wbase = wid * EW
    lane = lax.iota(jnp.int32, L)

    def zloop(i, _):
        zbuf[pl.ds(i * L, L)] = jnp.zeros((L,), jnp.float32)
        return 0

    lax.fori_loop(0, 5120 // L, zloop, 0)
    pltpu.sync_copy(zbuf, s_sp.at[pl.ds(sid * 5120, 5120)])
    dst_w[pl.ds(WB - 8, L)] = jnp.zeros((L,), jnp.int32)
    plsc.subcore_barrier()

    def win(w, _):
        base = wbase + w * WB
        for h in range(H):
            pltpu.sync_copy(alpha_hbm.at[pl.ds(h * EP + base, WB)],
                            a_w.at[pl.ds(h * 1024, WB)])
        pltpu.sync_copy(dst_hbm.at[pl.ds(base, WB)], dst_w.at[pl.ds(0, WB)])

        def group(g, _):
            valid = g * L + lane < WB
            d16 = dst_w[pl.ds(g * L, L)]
            for h in range(H):
                ex = jnp.exp(a_w[pl.ds(h * 1024 + g * L, L)])
                vals[pl.ds(g * (L * H) + h * L, L)] = jnp.where(valid, ex, 0.0)
                sidx[pl.ds(g * (L * H) + h * L, L)] = d16 * H + h
            return 0

        lax.fori_loop(0, NGB, group, 0)
        pltpu.sync_copy(vals, s_sp.at[sidx], add=True)
        return 0

    lax.fori_loop(0, EW // WB, win, 0)
    plsc.subcore_barrier()
    pltpu.sync_copy(s_sp.at[pl.ds(sid * 5120, 5120)],
                    out_hbm.at[pl.ds(cid * NHP + sid * 5120, 5120)])


@functools.partial(
    pl.kernel,
    out_type=jax.ShapeDtypeStruct((NC * NHP,), jnp.float32),
    mesh=_MESH,
    compiler_params=pltpu.CompilerParams(needs_layout_passes=False),
    scratch_types=[
        pltpu.VMEM((H * 1024,), jnp.float32),
        pltpu.VMEM((WB + 8, ), jnp.int32),
        pltpu.VMEM((NGB * L * H,), jnp.float32),
        pltpu.VMEM((NGB * L * H,), jnp.int32),
        pltpu.VMEM((5120,), jnp.float32),
        pltpu.VMEM_SHARED((NHP,), jnp.float32),
    ],
)
def _sc_segsum(alpha_hbm, dst_hbm, out_hbm, a_w, dst_w, vals, sidx, zbuf, s_sp):
    _seg_body(alpha_hbm, dst_hbm, out_hbm, a_w, dst_w, vals, sidx, zbuf, s_sp)


RT = 624
ZR = 104


def _agg_body(vt_hbm, alpha_hbm, src_hbm, dst_hbm, spart_hbm, out_hbm,
              tb0, tb1, src_w, dst_w, vi, si0, si1, sv0, sv1, a_w0, a_w1,
              at0, at1, v_w, sc, zb, out_sp, s_sp, semv, sems0, sems1):
    cid = lax.axis_index("c")
    sid = lax.axis_index("s")
    wid = sid * NC + cid
    wbase = wid * EW
    lane = lax.iota(jnp.int32, L)

    soff = sid * 5120
    pltpu.sync_copy(spart_hbm.at[pl.ds(soff, 5120)], tb0)
    pltpu.sync_copy(spart_hbm.at[pl.ds(NHP + soff, 5120)], tb1)

    def addv(j, _):
        o = j * L
        tb0[pl.ds(o, L)] = tb0[pl.ds(o, L)] + tb1[pl.ds(o, L)]
        return 0

    lax.fori_loop(0, 5120 // L, addv, 0)
    pltpu.sync_copy(tb0, s_sp.at[pl.ds(soff, 5120)])
    plsc.subcore_barrier()

    def zrow(i, _):
        for c4 in range(2 * D // L):
            zb[i, pl.ds(c4 * L, L)] = jnp.zeros((L,), jnp.float32)
        return 0

    lax.fori_loop(0, ZR, zrow, 0)

    for g in range(4):
        h0 = 2 * g
        h1 = 2 * g + 1

        def zcopy(i, _):
            pltpu.sync_copy(zb, out_sp.at[pl.ds(sid * RT + i * ZR, ZR)])
            return 0

        lax.fori_loop(0, RT // ZR, zcopy, 0)

        @pl.when(sid == 0)
        def _ztail():
            pltpu.sync_copy(zb.at[pl.ds(0, N - 16 * RT)],
                            out_sp.at[pl.ds(16 * RT, N - 16 * RT)])

        plsc.subcore_barrier()

        def win(w, _):
            base = wbase + w * WA
            pltpu.sync_copy(src_hbm.at[pl.ds(base, WA)], src_w)
            pltpu.sync_copy(dst_hbm.at[pl.ds(base, WA)], dst_w)

            def ibuild(g4, _):
                s16 = src_w[pl.ds(g4 * L, L)]
                d16 = dst_w[pl.ds(g4 * L, L)]
                vi[pl.ds(g4 * L, L)] = s16 + g * N
                si0[pl.ds(g4 * L, L)] = d16 * H + h0
                si1[pl.ds(g4 * L, L)] = d16 * H + h1
                return 0

            lax.fori_loop(0, WA // L, ibuild, 0)
            cv = pltpu.async_copy(vt_hbm.at[vi], v_w, semv)
            cs0 = pltpu.async_copy(s_sp.at[si0], sv0, sems0)
            cs1 = pltpu.async_copy(s_sp.at[si1], sv1, sems1)
            pltpu.sync_copy(alpha_hbm.at[pl.ds(h0 * EP + base, WA)], a_w0)
            pltpu.sync_copy(alpha_hbm.at[pl.ds(h1 * EP + base, WA)], a_w1)
            cs0.wait()
            cs1.wait()

            def attn(g4, _):
                valid = w * WA + g4 * L + lane < EW
                for (aw, svb, atb) in ((a_w0, sv0, at0), (a_w1, sv1, at1)):
                    ex = jnp.exp(aw[pl.ds(g4 * L, L)])
                    sv = svb[pl.ds(g4 * L, L)]
                    a = ex / (sv + 1e-16)
                    atb[pl.ds(g4 * L, L)] = jnp.where(valid, a, 0.0)
                return 0

            lax.fori_loop(0, WA // L, attn, 0)
            cv.wait()

            def scale(g4, _):
                av0 = at0[pl.ds(g4 * L, L)]
                av1 = at1[pl.ds(g4 * L, L)]
                for j in range(L):
                    e = g4 * L + j
                    b0 = jnp.full((L,), av0[j], jnp.float32)
                    b1 = jnp.full((L,), av1[j], jnp.float32)
                    for c4 in range(D // L):
                        o = c4 * L
                        sc[e, pl.ds(o, L)] = v_w[e, pl.ds(o, L)] * b0
                        sc[e, pl.ds(D + o, L)] = v_w[e, pl.ds(D + o, L)] * b1
                return 0

            lax.fori_loop(0, WA // L, scale, 0)
            pltpu.sync_copy(sc, out_sp.at[dst_w], add=True)
            return 0

        lax.fori_loop(0, NWIN, win, 0)
        plsc.subcore_barrier()
        pltpu.sync_copy(out_sp.at[pl.ds(sid * RT, RT)],
                        out_hbm.at[cid, g, pl.ds(sid * RT, RT), :])

        @pl.when(sid == 0)
        def _tail():
            pltpu.sync_copy(out_sp.at[pl.ds(16 * RT, N - 16 * RT)],
                            out_hbm.at[cid, g, pl.ds(16 * RT, N - 16 * RT), :])

        plsc.subcore_barrier()


@functools.partial(
    pl.kernel,
    out_type=jax.ShapeDtypeStruct((NC, 4, N, 2 * D), jnp.float32),
    mesh=_MESH,
    compiler_params=pltpu.CompilerParams(needs_layout_passes=False),
    scratch_types=[
        pltpu.VMEM((5120,), jnp.float32),
        pltpu.VMEM((5120,), jnp.float32),
        pltpu.VMEM((WA,), jnp.int32),
        pltpu.VMEM((WA,), jnp.int32),
        pltpu.VMEM((WA,), jnp.int32),
        pltpu.VMEM((WA,), jnp.int32),
        pltpu.VMEM((WA,), jnp.int32),
        pltpu.VMEM((WA,), jnp.float32),
        pltpu.VMEM((WA,), jnp.float32),
        pltpu.VMEM((WA,), jnp.float32),
        pltpu.VMEM((WA,), jnp.float32),
        pltpu.VMEM((WA,), jnp.float32),
        pltpu.VMEM((WA,), jnp.float32),
        pltpu.VMEM((WA, 2 * D), jnp.float32),
        pltpu.VMEM((WA, 2 * D), jnp.float32),
        pltpu.VMEM((ZR, 2 * D), jnp.float32),
        pltpu.VMEM_SHARED((N, 2 * D), jnp.float32),
        pltpu.VMEM_SHARED((NHP,), jnp.float32),
        pltpu.SemaphoreType.DMA,
        pltpu.SemaphoreType.DMA,
        pltpu.SemaphoreType.DMA,
    ],
)
def _sc_agg(vt_hbm, alpha_hbm, src_hbm, dst_hbm, spart_hbm, out_hbm,
            tb0, tb1, src_w, dst_w, vi, si0, si1, sv0, sv1, a_w0, a_w1,
            at0, at1, v_w, sc, zb, out_sp, s_sp, semv, sems0, sems1):
    _agg_body(vt_hbm, alpha_hbm, src_hbm, dst_hbm, spart_hbm, out_hbm,
              tb0, tb1, src_w, dst_w, vi, si0, si1, sv0, sv1, a_w0, a_w1,
              at0, at1, v_w, sc, zb, out_sp, s_sp, semv, sems0, sems1)


def _combine_body(p0_ref, p1_ref, x_ref, ws_ref, bs_ref, o_ref):
    t = jnp.sum(p0_ref[...], axis=0) + jnp.sum(p1_ref[...], axis=0)
    acc = t[:, :D] + t[:, D:]
    o_ref[...] = acc * jnp.float32(1.0 / H) + jnp.dot(
        x_ref[...], ws_ref[...], preferred_element_type=jnp.float32) + bs_ref[...]


def _combine(p0, p1, x, Wskip, bskip):
    grid = (N // BN,)
    return pl.pallas_call(
        _combine_body,
        grid=grid,
        in_specs=[
            pl.BlockSpec((4, BN, 2 * D), lambda i: (0, i, 0)),
            pl.BlockSpec((4, BN, 2 * D), lambda i: (0, i, 0)),
            pl.BlockSpec((BN, F), lambda i: (i, 0)),
            pl.BlockSpec((F, D), lambda i: (0, 0)),
            pl.BlockSpec((1, D), lambda i: (0, 0)),
        ],
        out_specs=pl.BlockSpec((BN, D), lambda i: (i, 0)),
        out_shape=jax.ShapeDtypeStruct((N, D), jnp.float32),
    )(p0, p1, x, Wskip, bskip.reshape(1, D))


def kernel(x, edge_index, Wq, bq, Wk, bk, Wv, bv, Wskip, bskip):
    src = edge_index[0]
    dst = edge_index[1]
    srcp = jnp.concatenate([src, jnp.zeros((EP - E,), jnp.int32)])
    dstp = jnp.concatenate([dst, jnp.zeros((EP - E,), jnp.int32)])
    q, k, vt = _projections(x, Wq, bq, Wk, bk, Wv, bv)

    alpha_flat = _sc_alpha(q, k, srcp, dstp)
    spart = _sc_segsum(alpha_flat, dstp)
    outp = _sc_agg(vt, alpha_flat, srcp, dstp, spart)
    return _combine(outp[0], outp[1], x, Wskip, bskip)


# speedup vs baseline: 13.8437x; 1.0440x over previous
import functools

import jax
import jax.numpy as jnp
from jax import lax
from jax.experimental import pallas as pl
from jax.experimental.pallas import tpu as pltpu
from jax.experimental.pallas import tpu_sc as plsc

N = 10000
E = 160000
F = 256
H = 8
D = 64
HD = H * D
BN = 1000
SCALE = 0.125

NC = 2
NS = 16
L = 16
NW = NC * NS
EW = E // NW
WA = 64
NWIN = (EW + WA - 1) // WA
WQ = 112
NWINQ = (EW + WQ - 1) // WQ
EP = E + 2 * WQ

_MESH = plsc.VectorSubcoreMesh(core_axis_name="c", subcore_axis_name="s",
                               num_cores=NC, num_subcores=NS)


def _worker_id():
    return lax.axis_index("s") * NC + lax.axis_index("c")


def _alpha_body(q_hbm, k_hbm, src_hbm, dst_hbm, alpha_hbm,
                sidx, didx, qd, ks, at, semq, semk, semw):
    wid = _worker_id()
    wbase = wid * EW
    lane = lax.iota(jnp.int32, L)

    def win(j, _):
        base = wbase + j * WQ
        pltpu.sync_copy(src_hbm.at[pl.ds(base, WQ)], sidx)
        pltpu.sync_copy(dst_hbm.at[pl.ds(base, WQ)], didx)
        cq = pltpu.async_copy(q_hbm.at[didx], qd, semq)
        ck = pltpu.async_copy(k_hbm.at[sidx], ks, semk)
        cq.wait()
        ck.wait()

        @pl.when(j > 0)
        def _drain():
            for h in range(H):
                pltpu.make_async_copy(
                    at.at[pl.ds(h * 1024, WQ)],
                    alpha_hbm.at[pl.ds(h * EP + base - WQ, WQ)],
                    semw).wait()

        def group(g, _):
            def edge(jj, accs):
                e = g * L + jj
                new = []
                for h in range(H):
                    o = h * D
                    p = qd[e, pl.ds(o, L)] * ks[e, pl.ds(o, L)]
                    p = p + qd[e, pl.ds(o + L, L)] * ks[e, pl.ds(o + L, L)]
                    p = (p + qd[e, pl.ds(o + 2 * L, L)] *
                         ks[e, pl.ds(o + 2 * L, L)])
                    p = (p + qd[e, pl.ds(o + 3 * L, L)] *
                         ks[e, pl.ds(o + 3 * L, L)])
                    s = jnp.sum(p) * SCALE
                    new.append(jnp.where(lane == jj, s, accs[h]))
                return tuple(new)

            accs = lax.fori_loop(0, L, edge,
                                 tuple(jnp.zeros((L,), jnp.float32)
                                       for _ in range(H)))
            for h in range(H):
                at[pl.ds(h * 1024 + g * L, L)] = accs[h]
            return 0

        lax.fori_loop(0, WQ // L, group, 0)
        for h in range(H):
            pltpu.async_copy(at.at[pl.ds(h * 1024, WQ)],
                             alpha_hbm.at[pl.ds(h * EP + base, WQ)], semw)
        return 0

    lax.fori_loop(0, NWINQ, win, 0)
    for h in range(H):
        pltpu.make_async_copy(
            at.at[pl.ds(h * 1024, WQ)],
            alpha_hbm.at[pl.ds(h * EP + wbase + (NWINQ - 1) * WQ, WQ)],
            semw).wait()


@functools.partial(
    pl.kernel,
    out_type=jax.ShapeDtypeStruct((H * EP,), jnp.float32),
    mesh=_MESH,
    compiler_params=pltpu.CompilerParams(needs_layout_passes=False),
    scratch_types=[
        pltpu.VMEM((WQ,), jnp.int32),
        pltpu.VMEM((WQ,), jnp.int32),
        pltpu.VMEM((WQ, HD), jnp.float32),
        pltpu.VMEM((WQ, HD), jnp.float32),
        pltpu.VMEM((H * 1024,), jnp.float32),
        pltpu.SemaphoreType.DMA,
        pltpu.SemaphoreType.DMA,
        pltpu.SemaphoreType.DMA,
    ],
)
def _sc_alpha(q_hbm, k_hbm, src_hbm, dst_hbm, alpha_hbm,
              sidx, didx, qd, ks, at, semq, semk, semw):
    _alpha_body(q_hbm, k_hbm, src_hbm, dst_hbm, alpha_hbm,
                sidx, didx, qd, ks, at, semq, semk, semw)


def _proj_body(x_ref, wq_ref, bq_ref, wk_ref, bk_ref, wv_ref, bv_ref,
               q_ref, k_ref, vt_ref):
    xb = x_ref[...]
    q_ref[...] = jnp.dot(xb, wq_ref[...],
                         preferred_element_type=jnp.float32) + bq_ref[...]
    k_ref[...] = jnp.dot(xb, wk_ref[...],
                         preferred_element_type=jnp.float32) + bk_ref[...]
    vt_ref[...] = jnp.dot(xb, wv_ref[...],
                          preferred_element_type=jnp.float32) + bv_ref[...]


def _projections(x, Wq, bq, Wk, bk, Wv, bv):
    grid = (N // BN, H // 2)
    w_spec = pl.BlockSpec((F, 2 * D), lambda i, h: (0, h))
    b_spec = pl.BlockSpec((1, 2 * D), lambda i, h: (0, h))
    return pl.pallas_call(
        _proj_body,
        grid=grid,
        in_specs=[
            pl.BlockSpec((BN, F), lambda i, h: (i, 0)),
            w_spec, b_spec, w_spec, b_spec, w_spec, b_spec,
        ],
        out_specs=[
            pl.BlockSpec((BN, 2 * D), lambda i, h: (i, h)),
            pl.BlockSpec((BN, 2 * D), lambda i, h: (i, h)),
            pl.BlockSpec((BN, 2 * D), lambda i, h: (h * (N // BN) + i, 0)),
        ],
        out_shape=[
            jax.ShapeDtypeStruct((N, HD), jnp.float32),
            jax.ShapeDtypeStruct((N, HD), jnp.float32),
            jax.ShapeDtypeStruct((4 * N, 2 * D), jnp.float32),
        ],
    )(x, Wq, bq.reshape(1, HD), Wk, bk.reshape(1, HD), Wv, bv.reshape(1, HD))


NHP = 81920
WB = 1000
NGB = WB // L + 1


def _seg_body(alpha_hbm, dst_hbm, out_hbm, a_w, dst_w, vals, sidx, zbuf, s_sp):
    cid = lax.axis_index("c")
    sid = lax.axis_index("s")
    wid = sid * NC + cid
    wbase = wid * EW
    lane = lax.iota(jnp.int32, L)

    def zloop(i, _):
        zbuf[pl.ds(i * L, L)] = jnp.zeros((L,), jnp.float32)
        return 0

    lax.fori_loop(0, 5120 // L, zloop, 0)
    pltpu.sync_copy(zbuf, s_sp.at[pl.ds(sid * 5120, 5120)])
    dst_w[pl.ds(WB - 8, L)] = jnp.zeros((L,), jnp.int32)
    plsc.subcore_barrier()

    def win(w, _):
        base = wbase + w * WB
        for h in range(H):
            pltpu.sync_copy(alpha_hbm.at[pl.ds(h * EP + base, WB)],
                            a_w.at[pl.ds(h * 1024, WB)])
        pltpu.sync_copy(dst_hbm.at[pl.ds(base, WB)], dst_w.at[pl.ds(0, WB)])

        def group(g, _):
            valid = g * L + lane < WB
            d16 = dst_w[pl.ds(g * L, L)]
            for h in range(H):
                ex = jnp.exp(a_w[pl.ds(h * 1024 + g * L, L)])
                vals[pl.ds(g * (L * H) + h * L, L)] = jnp.where(valid, ex, 0.0)
                sidx[pl.ds(g * (L * H) + h * L, L)] = d16 * H + h
            return 0

        lax.fori_loop(0, NGB, group, 0)
        pltpu.sync_copy(vals, s_sp.at[sidx], add=True)
        return 0

    lax.fori_loop(0, EW // WB, win, 0)
    plsc.subcore_barrier()
    pltpu.sync_copy(s_sp.at[pl.ds(sid * 5120, 5120)],
                    out_hbm.at[pl.ds(cid * NHP + sid * 5120, 5120)])


@functools.partial(
    pl.kernel,
    out_type=jax.ShapeDtypeStruct((NC * NHP,), jnp.float32),
    mesh=_MESH,
    compiler_params=pltpu.CompilerParams(needs_layout_passes=False),
    scratch_types=[
        pltpu.VMEM((H * 1024,), jnp.float32),
        pltpu.VMEM((WB + 8, ), jnp.int32),
        pltpu.VMEM((NGB * L * H,), jnp.float32),
        pltpu.VMEM((NGB * L * H,), jnp.int32),
        pltpu.VMEM((5120,), jnp.float32),
        pltpu.VMEM_SHARED((NHP,), jnp.float32),
    ],
)
def _sc_segsum(alpha_hbm, dst_hbm, out_hbm, a_w, dst_w, vals, sidx, zbuf, s_sp):
    _seg_body(alpha_hbm, dst_hbm, out_hbm, a_w, dst_w, vals, sidx, zbuf, s_sp)


RT = 624
ZR = 104


def _agg_body(vt_hbm, alpha_hbm, src_hbm, dst_hbm, spart_hbm, out_hbm,
              tb0, tb1, src_w, dst_w, vi, si0, si1, sv0, sv1, a_w0, a_w1,
              at0, at1, v_w, sc, zb, out_sp, s_sp, semv, sems0, sems1):
    cid = lax.axis_index("c")
    sid = lax.axis_index("s")
    wid = sid * NC + cid
    wbase = wid * EW
    lane = lax.iota(jnp.int32, L)

    soff = sid * 5120
    pltpu.sync_copy(spart_hbm.at[pl.ds(soff, 5120)], tb0)
    pltpu.sync_copy(spart_hbm.at[pl.ds(NHP + soff, 5120)], tb1)

    def addv(j, _):
        o = j * L
        tb0[pl.ds(o, L)] = tb0[pl.ds(o, L)] + tb1[pl.ds(o, L)]
        return 0

    lax.fori_loop(0, 5120 // L, addv, 0)
    pltpu.sync_copy(tb0, s_sp.at[pl.ds(soff, 5120)])
    plsc.subcore_barrier()

    def zrow(i, _):
        for c4 in range(2 * D // L):
            zb[i, pl.ds(c4 * L, L)] = jnp.zeros((L,), jnp.float32)
        return 0

    lax.fori_loop(0, ZR, zrow, 0)

    for g in range(4):
        h0 = 2 * g
        h1 = 2 * g + 1

        def zcopy(i, _):
            pltpu.sync_copy(zb, out_sp.at[pl.ds(sid * RT + i * ZR, ZR)])
            return 0

        lax.fori_loop(0, RT // ZR, zcopy, 0)

        @pl.when(sid == 0)
        def _ztail():
            pltpu.sync_copy(zb.at[pl.ds(0, N - 16 * RT)],
                            out_sp.at[pl.ds(16 * RT, N - 16 * RT)])

        plsc.subcore_barrier()

        def win(w, _):
            base = wbase + w * WA
            pltpu.sync_copy(src_hbm.at[pl.ds(base, WA)], src_w)
            pltpu.sync_copy(dst_hbm.at[pl.ds(base, WA)], dst_w)

            def ibuild(g4, _):
                s16 = src_w[pl.ds(g4 * L, L)]
                d16 = dst_w[pl.ds(g4 * L, L)]
                vi[pl.ds(g4 * L, L)] = s16 + g * N
                si0[pl.ds(g4 * L, L)] = d16 * H + h0
                si1[pl.ds(g4 * L, L)] = d16 * H + h1
                return 0

            lax.fori_loop(0, WA // L, ibuild, 0)
            cv = pltpu.async_copy(vt_hbm.at[vi], v_w, semv)
            cs0 = pltpu.async_copy(s_sp.at[si0], sv0, sems0)
            cs1 = pltpu.async_copy(s_sp.at[si1], sv1, sems1)
            pltpu.sync_copy(alpha_hbm.at[pl.ds(h0 * EP + base, WA)], a_w0)
            pltpu.sync_copy(alpha_hbm.at[pl.ds(h1 * EP + base, WA)], a_w1)
            cs0.wait()
            cs1.wait()

            def attn(g4, _):
                valid = w * WA + g4 * L + lane < EW
                for (aw, svb, atb) in ((a_w0, sv0, at0), (a_w1, sv1, at1)):
                    ex = jnp.exp(aw[pl.ds(g4 * L, L)])
                    sv = svb[pl.ds(g4 * L, L)]
                    a = ex / (sv + 1e-16)
                    atb[pl.ds(g4 * L, L)] = jnp.where(valid, a, 0.0)
                return 0

            lax.fori_loop(0, WA // L, attn, 0)
            cv.wait()

            def scale(g4, _):
                av0 = at0[pl.ds(g4 * L, L)]
                av1 = at1[pl.ds(g4 * L, L)]
                for j in range(L):
                    e = g4 * L + j
                    b0 = jnp.full((L,), av0[j], jnp.float32)
                    b1 = jnp.full((L,), av1[j], jnp.float32)
                    for c4 in range(D // L):
                        o = c4 * L
                        sc[e, pl.ds(o, L)] = v_w[e, pl.ds(o, L)] * b0
                        sc[e, pl.ds(D + o, L)] = v_w[e, pl.ds(D + o, L)] * b1
                return 0

            lax.fori_loop(0, WA // L, scale, 0)
            pltpu.sync_copy(sc, out_sp.at[dst_w], add=True)
            return 0

        lax.fori_loop(0, NWIN, win, 0)
        plsc.subcore_barrier()
        pltpu.sync_copy(out_sp.at[pl.ds(sid * RT, RT)],
                        out_hbm.at[cid, g, pl.ds(sid * RT, RT), :])

        @pl.when(sid == 0)
        def _tail():
            pltpu.sync_copy(out_sp.at[pl.ds(16 * RT, N - 16 * RT)],
                            out_hbm.at[cid, g, pl.ds(16 * RT, N - 16 * RT), :])

        plsc.subcore_barrier()


@functools.partial(
    pl.kernel,
    out_type=jax.ShapeDtypeStruct((NC, 4, N, 2 * D), jnp.float32),
    mesh=_MESH,
    compiler_params=pltpu.CompilerParams(needs_layout_passes=False),
    scratch_types=[
        pltpu.VMEM((5120,), jnp.float32),
        pltpu.VMEM((5120,), jnp.float32),
        pltpu.VMEM((WA,), jnp.int32),
        pltpu.VMEM((WA,), jnp.int32),
        pltpu.VMEM((WA,), jnp.int32),
        pltpu.VMEM((WA,), jnp.int32),
        pltpu.VMEM((WA,), jnp.int32),
        pltpu.VMEM((WA,), jnp.float32),
        pltpu.VMEM((WA,), jnp.float32),
        pltpu.VMEM((WA,), jnp.float32),
        pltpu.VMEM((WA,), jnp.float32),
        pltpu.VMEM((WA,), jnp.float32),
        pltpu.VMEM((WA,), jnp.float32),
        pltpu.VMEM((WA, 2 * D), jnp.float32),
        pltpu.VMEM((WA, 2 * D), jnp.float32),
        pltpu.VMEM((ZR, 2 * D), jnp.float32),
        pltpu.VMEM_SHARED((N, 2 * D), jnp.float32),
        pltpu.VMEM_SHARED((NHP,), jnp.float32),
        pltpu.SemaphoreType.DMA,
        pltpu.SemaphoreType.DMA,
        pltpu.SemaphoreType.DMA,
    ],
)
def _sc_agg(vt_hbm, alpha_hbm, src_hbm, dst_hbm, spart_hbm, out_hbm,
            tb0, tb1, src_w, dst_w, vi, si0, si1, sv0, sv1, a_w0, a_w1,
            at0, at1, v_w, sc, zb, out_sp, s_sp, semv, sems0, sems1):
    _agg_body(vt_hbm, alpha_hbm, src_hbm, dst_hbm, spart_hbm, out_hbm,
              tb0, tb1, src_w, dst_w, vi, si0, si1, sv0, sv1, a_w0, a_w1,
              at0, at1, v_w, sc, zb, out_sp, s_sp, semv, sems0, sems1)


def _combine_body(p0_ref, p1_ref, x_ref, ws_ref, bs_ref, o_ref):
    t = jnp.sum(p0_ref[...], axis=0) + jnp.sum(p1_ref[...], axis=0)
    acc = t[:, :D] + t[:, D:]
    o_ref[...] = acc * jnp.float32(1.0 / H) + jnp.dot(
        x_ref[...], ws_ref[...], preferred_element_type=jnp.float32) + bs_ref[...]


def _combine(p0, p1, x, Wskip, bskip):
    grid = (N // BN,)
    return pl.pallas_call(
        _combine_body,
        grid=grid,
        in_specs=[
            pl.BlockSpec((4, BN, 2 * D), lambda i: (0, i, 0)),
            pl.BlockSpec((4, BN, 2 * D), lambda i: (0, i, 0)),
            pl.BlockSpec((BN, F), lambda i: (i, 0)),
            pl.BlockSpec((F, D), lambda i: (0, 0)),
            pl.BlockSpec((1, D), lambda i: (0, 0)),
        ],
        out_specs=pl.BlockSpec((BN, D), lambda i: (i, 0)),
        out_shape=jax.ShapeDtypeStruct((N, D), jnp.float32),
    )(p0, p1, x, Wskip, bskip.reshape(1, D))


def kernel(x, edge_index, Wq, bq, Wk, bk, Wv, bv, Wskip, bskip):
    src = edge_index[0]
    dst = edge_index[1]
    srcp = jnp.concatenate([src, jnp.zeros((EP - E,), jnp.int32)])
    dstp = jnp.concatenate([dst, jnp.zeros((EP - E,), jnp.int32)])
    q, k, vt = _projections(x, Wq, bq, Wk, bk, Wv, bv)

    alpha_flat = _sc_alpha(q, k, srcp, dstp)
    spart = _sc_segsum(alpha_flat, dstp)
    outp = _sc_agg(vt, alpha_flat, srcp, dstp, spart)
    return _combine(outp[0], outp[1], x, Wskip, bskip)


# speedup vs baseline: 15.0715x; 1.0887x over previous
import functools

import jax
import jax.numpy as jnp
from jax import lax
from jax.experimental import pallas as pl
from jax.experimental.pallas import tpu as pltpu
from jax.experimental.pallas import tpu_sc as plsc

N = 10000
E = 160000
F = 256
H = 8
D = 64
HD = H * D
BN = 1000
SCALE = 0.125

NC = 2
NS = 16
L = 16
NW = NC * NS
EW = E // NW
WA = 64
NWIN = (EW + WA - 1) // WA
WQ = 112
NWINQ = (EW + WQ - 1) // WQ
EP = E + 2 * WQ

_MESH = plsc.VectorSubcoreMesh(core_axis_name="c", subcore_axis_name="s",
                               num_cores=NC, num_subcores=NS)


def _worker_id():
    return lax.axis_index("s") * NC + lax.axis_index("c")


def _alpha_body(q_hbm, k_hbm, src_hbm, dst_hbm, alpha_hbm,
                sidx, didx, qd, ks, at, semq, semk, semw):
    wid = _worker_id()
    wbase = wid * EW
    lane = lax.iota(jnp.int32, L)

    def win(j, _):
        base = wbase + j * WQ
        pltpu.sync_copy(src_hbm.at[pl.ds(base, WQ)], sidx)
        pltpu.sync_copy(dst_hbm.at[pl.ds(base, WQ)], didx)
        cq = pltpu.async_copy(q_hbm.at[didx], qd, semq)
        ck = pltpu.async_copy(k_hbm.at[sidx], ks, semk)
        cq.wait()
        ck.wait()

        @pl.when(j > 0)
        def _drain():
            for h in range(H):
                pltpu.make_async_copy(
                    at.at[pl.ds(h * 1024, WQ)],
                    alpha_hbm.at[pl.ds(h * EP + base - WQ, WQ)],
                    semw).wait()

        def group(g, _):
            def edge(jj, accs):
                e = g * L + jj
                new = []
                for h in range(H):
                    o = h * D
                    p = qd[e, pl.ds(o, L)] * ks[e, pl.ds(o, L)]
                    p = p + qd[e, pl.ds(o + L, L)] * ks[e, pl.ds(o + L, L)]
                    p = (p + qd[e, pl.ds(o + 2 * L, L)] *
                         ks[e, pl.ds(o + 2 * L, L)])
                    p = (p + qd[e, pl.ds(o + 3 * L, L)] *
                         ks[e, pl.ds(o + 3 * L, L)])
                    s = jnp.sum(p) * SCALE
                    new.append(jnp.where(lane == jj, s, accs[h]))
                return tuple(new)

            accs = lax.fori_loop(0, L, edge,
                                 tuple(jnp.zeros((L,), jnp.float32)
                                       for _ in range(H)))
            for h in range(H):
                at[pl.ds(h * 1024 + g * L, L)] = accs[h]
            return 0

        lax.fori_loop(0, WQ // L, group, 0)
        for h in range(H):
            pltpu.async_copy(at.at[pl.ds(h * 1024, WQ)],
                             alpha_hbm.at[pl.ds(h * EP + base, WQ)], semw)
        return 0

    lax.fori_loop(0, NWINQ, win, 0)
    for h in range(H):
        pltpu.make_async_copy(
            at.at[pl.ds(h * 1024, WQ)],
            alpha_hbm.at[pl.ds(h * EP + wbase + (NWINQ - 1) * WQ, WQ)],
            semw).wait()


@functools.partial(
    pl.kernel,
    out_type=jax.ShapeDtypeStruct((H * EP,), jnp.float32),
    mesh=_MESH,
    compiler_params=pltpu.CompilerParams(needs_layout_passes=False),
    scratch_types=[
        pltpu.VMEM((WQ,), jnp.int32),
        pltpu.VMEM((WQ,), jnp.int32),
        pltpu.VMEM((WQ, HD), jnp.float32),
        pltpu.VMEM((WQ, HD), jnp.float32),
        pltpu.VMEM((H * 1024,), jnp.float32),
        pltpu.SemaphoreType.DMA,
        pltpu.SemaphoreType.DMA,
        pltpu.SemaphoreType.DMA,
    ],
)
def _sc_alpha(q_hbm, k_hbm, src_hbm, dst_hbm, alpha_hbm,
              sidx, didx, qd, ks, at, semq, semk, semw):
    _alpha_body(q_hbm, k_hbm, src_hbm, dst_hbm, alpha_hbm,
                sidx, didx, qd, ks, at, semq, semk, semw)


def _proj_body(x_ref, wq_ref, bq_ref, wk_ref, bk_ref, wv_ref, bv_ref,
               q_ref, k_ref, vt_ref):
    xb = x_ref[...]
    q_ref[...] = jnp.dot(xb, wq_ref[...],
                         preferred_element_type=jnp.float32) + bq_ref[...]
    k_ref[...] = jnp.dot(xb, wk_ref[...],
                         preferred_element_type=jnp.float32) + bk_ref[...]
    vt_ref[...] = jnp.dot(xb, wv_ref[...],
                          preferred_element_type=jnp.float32) + bv_ref[...]


def _projections(x, Wq, bq, Wk, bk, Wv, bv):
    grid = (N // BN, H // 2)
    w_spec = pl.BlockSpec((F, 2 * D), lambda i, h: (0, h))
    b_spec = pl.BlockSpec((1, 2 * D), lambda i, h: (0, h))
    return pl.pallas_call(
        _proj_body,
        grid=grid,
        in_specs=[
            pl.BlockSpec((BN, F), lambda i, h: (i, 0)),
            w_spec, b_spec, w_spec, b_spec, w_spec, b_spec,
        ],
        out_specs=[
            pl.BlockSpec((BN, 2 * D), lambda i, h: (i, h)),
            pl.BlockSpec((BN, 2 * D), lambda i, h: (i, h)),
            pl.BlockSpec((BN, 2 * D), lambda i, h: (h * (N // BN) + i, 0)),
        ],
        out_shape=[
            jax.ShapeDtypeStruct((N, HD), jnp.float32),
            jax.ShapeDtypeStruct((N, HD), jnp.float32),
            jax.ShapeDtypeStruct((4 * N, 2 * D), jnp.float32),
        ],
    )(x, Wq, bq.reshape(1, HD), Wk, bk.reshape(1, HD), Wv, bv.reshape(1, HD))


NHP = 81920
WB = 1000
NGB = WB // L + 1


def _seg_body(alpha_hbm, dst_hbm, out_hbm, a_w, dst_w, vals, sidx, zbuf, s_sp):
    cid = lax.axis_index("c")
    sid = lax.axis_index("s")
    wid = sid * NC + cid
    wbase = wid * EW
    lane = lax.iota(jnp.int32, L)

    def zloop(i, _):
        zbuf[pl.ds(i * L, L)] = jnp.zeros((L,), jnp.float32)
        return 0

    lax.fori_loop(0, 5120 // L, zloop, 0)
    pltpu.sync_copy(zbuf, s_sp.at[pl.ds(sid * 5120, 5120)])
    dst_w[pl.ds(WB - 8, L)] = jnp.zeros((L,), jnp.int32)
    plsc.subcore_barrier()

    def win(w, _):
        base = wbase + w * WB
        for h in range(H):
            pltpu.sync_copy(alpha_hbm.at[pl.ds(h * EP + base, WB)],
                            a_w.at[pl.ds(h * 1024, WB)])
        pltpu.sync_copy(dst_hbm.at[pl.ds(base, WB)], dst_w.at[pl.ds(0, WB)])

        def group(g, _):
            valid = g * L + lane < WB
            d16 = dst_w[pl.ds(g * L, L)]
            for h in range(H):
                ex = jnp.exp(a_w[pl.ds(h * 1024 + g * L, L)])
                vals[pl.ds(g * (L * H) + h * L, L)] = jnp.where(valid, ex, 0.0)
                sidx[pl.ds(g * (L * H) + h * L, L)] = d16 * H + h
            return 0

        lax.fori_loop(0, NGB, group, 0)
        pltpu.sync_copy(vals, s_sp.at[sidx], add=True)
        return 0

    lax.fori_loop(0, EW // WB, win, 0)
    plsc.subcore_barrier()
    pltpu.sync_copy(s_sp.at[pl.ds(sid * 5120, 5120)],
                    out_hbm.at[pl.ds(cid * NHP + sid * 5120, 5120)])


@functools.partial(
    pl.kernel,
    out_type=jax.ShapeDtypeStruct((NC * NHP,), jnp.float32),
    mesh=_MESH,
    compiler_params=pltpu.CompilerParams(needs_layout_passes=False),
    scratch_types=[
        pltpu.VMEM((H * 1024,), jnp.float32),
        pltpu.VMEM((WB + 8, ), jnp.int32),
        pltpu.VMEM((NGB * L * H,), jnp.float32),
        pltpu.VMEM((NGB * L * H,), jnp.int32),
        pltpu.VMEM((5120,), jnp.float32),
        pltpu.VMEM_SHARED((NHP,), jnp.float32),
    ],
)
def _sc_segsum(alpha_hbm, dst_hbm, out_hbm, a_w, dst_w, vals, sidx, zbuf, s_sp):
    _seg_body(alpha_hbm, dst_hbm, out_hbm, a_w, dst_w, vals, sidx, zbuf, s_sp)


RT = 624
ZR = 104


def _agg_body(vt_hbm, alpha_hbm, src_hbm, dst_hbm, spart_hbm, out_hbm,
              tb0, tb1, src_w, dst_w, vi, si0, si1, sv0, sv1, a_w0, a_w1,
              at0, at1, v_w, sc, zb, out_sp, s_sp, semv, sems0, sems1, semi):
    cid = lax.axis_index("c")
    sid = lax.axis_index("s")
    wid = sid * NC + cid
    wbase = wid * EW
    lane = lax.iota(jnp.int32, L)

    soff = sid * 5120
    pltpu.sync_copy(spart_hbm.at[pl.ds(soff, 5120)], tb0)
    pltpu.sync_copy(spart_hbm.at[pl.ds(NHP + soff, 5120)], tb1)

    def addv(j, _):
        o = j * L
        tb0[pl.ds(o, L)] = tb0[pl.ds(o, L)] + tb1[pl.ds(o, L)]
        return 0

    lax.fori_loop(0, 5120 // L, addv, 0)
    pltpu.sync_copy(tb0, s_sp.at[pl.ds(soff, 5120)])
    plsc.subcore_barrier()

    def zrow(i, _):
        for c4 in range(2 * D // L):
            zb[i, pl.ds(c4 * L, L)] = jnp.zeros((L,), jnp.float32)
        return 0

    lax.fori_loop(0, ZR, zrow, 0)

    for g in range(4):
        h0 = 2 * g
        h1 = 2 * g + 1

        def zcopy(i, _):
            pltpu.sync_copy(zb, out_sp.at[pl.ds(sid * RT + i * ZR, ZR)])
            return 0

        lax.fori_loop(0, RT // ZR, zcopy, 0)

        @pl.when(sid == 0)
        def _ztail():
            pltpu.sync_copy(zb.at[pl.ds(0, N - 16 * RT)],
                            out_sp.at[pl.ds(16 * RT, N - 16 * RT)])

        plsc.subcore_barrier()

        def win(w, _):
            base = wbase + w * WA
            pltpu.async_copy(src_hbm.at[pl.ds(base, WA)], src_w, semi)
            pltpu.async_copy(dst_hbm.at[pl.ds(base, WA)], dst_w, semi)
            pltpu.make_async_copy(src_hbm.at[pl.ds(base, WA)], src_w,
                                  semi).wait()
            pltpu.make_async_copy(dst_hbm.at[pl.ds(base, WA)], dst_w,
                                  semi).wait()

            def ibuild(g4, _):
                s16 = src_w[pl.ds(g4 * L, L)]
                d16 = dst_w[pl.ds(g4 * L, L)]
                vi[pl.ds(g4 * L, L)] = s16 + g * N
                si0[pl.ds(g4 * L, L)] = d16 * H + h0
                si1[pl.ds(g4 * L, L)] = d16 * H + h1
                return 0

            lax.fori_loop(0, WA // L, ibuild, 0)
            cv = pltpu.async_copy(vt_hbm.at[vi], v_w, semv)
            cs0 = pltpu.async_copy(s_sp.at[si0], sv0, sems0)
            cs1 = pltpu.async_copy(s_sp.at[si1], sv1, sems1)
            pltpu.async_copy(alpha_hbm.at[pl.ds(h0 * EP + base, WA)], a_w0,
                             semi)
            pltpu.async_copy(alpha_hbm.at[pl.ds(h1 * EP + base, WA)], a_w1,
                             semi)
            cs0.wait()
            cs1.wait()
            pltpu.make_async_copy(alpha_hbm.at[pl.ds(h0 * EP + base, WA)],
                                  a_w0, semi).wait()
            pltpu.make_async_copy(alpha_hbm.at[pl.ds(h1 * EP + base, WA)],
                                  a_w1, semi).wait()

            def attn(g4, _):
                valid = w * WA + g4 * L + lane < EW
                for (aw, svb, atb) in ((a_w0, sv0, at0), (a_w1, sv1, at1)):
                    ex = jnp.exp(aw[pl.ds(g4 * L, L)])
                    sv = svb[pl.ds(g4 * L, L)]
                    a = ex / (sv + 1e-16)
                    atb[pl.ds(g4 * L, L)] = jnp.where(valid, a, 0.0)
                return 0

            lax.fori_loop(0, WA // L, attn, 0)
            cv.wait()

            def scale(g4, _):
                av0 = at0[pl.ds(g4 * L, L)]
                av1 = at1[pl.ds(g4 * L, L)]
                for j in range(L):
                    e = g4 * L + j
                    b0 = jnp.full((L,), av0[j], jnp.float32)
                    b1 = jnp.full((L,), av1[j], jnp.float32)
                    for c4 in range(D // L):
                        o = c4 * L
                        sc[e, pl.ds(o, L)] = v_w[e, pl.ds(o, L)] * b0
                        sc[e, pl.ds(D + o, L)] = v_w[e, pl.ds(D + o, L)] * b1
                return 0

            lax.fori_loop(0, WA // L, scale, 0)
            pltpu.sync_copy(sc, out_sp.at[dst_w], add=True)
            return 0

        lax.fori_loop(0, NWIN, win, 0)
        plsc.subcore_barrier()
        pltpu.sync_copy(out_sp.at[pl.ds(sid * RT, RT)],
                        out_hbm.at[cid, g, pl.ds(sid * RT, RT), :])

        @pl.when(sid == 0)
        def _tail():
            pltpu.sync_copy(out_sp.at[pl.ds(16 * RT, N - 16 * RT)],
                            out_hbm.at[cid, g, pl.ds(16 * RT, N - 16 * RT), :])

        plsc.subcore_barrier()


@functools.partial(
    pl.kernel,
    out_type=jax.ShapeDtypeStruct((NC, 4, N, 2 * D), jnp.float32),
    mesh=_MESH,
    compiler_params=pltpu.CompilerParams(needs_layout_passes=False),
    scratch_types=[
        pltpu.VMEM((5120,), jnp.float32),
        pltpu.VMEM((5120,), jnp.float32),
        pltpu.VMEM((WA,), jnp.int32),
        pltpu.VMEM((WA,), jnp.int32),
        pltpu.VMEM((WA,), jnp.int32),
        pltpu.VMEM((WA,), jnp.int32),
        pltpu.VMEM((WA,), jnp.int32),
        pltpu.VMEM((WA,), jnp.float32),
        pltpu.VMEM((WA,), jnp.float32),
        pltpu.VMEM((WA,), jnp.float32),
        pltpu.VMEM((WA,), jnp.float32),
        pltpu.VMEM((WA,), jnp.float32),
        pltpu.VMEM((WA,), jnp.float32),
        pltpu.VMEM((WA, 2 * D), jnp.float32),
        pltpu.VMEM((WA, 2 * D), jnp.float32),
        pltpu.VMEM((ZR, 2 * D), jnp.float32),
        pltpu.VMEM_SHARED((N, 2 * D), jnp.float32),
        pltpu.VMEM_SHARED((NHP,), jnp.float32),
        pltpu.SemaphoreType.DMA,
        pltpu.SemaphoreType.DMA,
        pltpu.SemaphoreType.DMA,
        pltpu.SemaphoreType.DMA,
    ],
)
def _sc_agg(vt_hbm, alpha_hbm, src_hbm, dst_hbm, spart_hbm, out_hbm,
            tb0, tb1, src_w, dst_w, vi, si0, si1, sv0, sv1, a_w0, a_w1,
            at0, at1, v_w, sc, zb, out_sp, s_sp, semv, sems0, sems1, semi):
    _agg_body(vt_hbm, alpha_hbm, src_hbm, dst_hbm, spart_hbm, out_hbm,
              tb0, tb1, src_w, dst_w, vi, si0, si1, sv0, sv1, a_w0, a_w1,
              at0, at1, v_w, sc, zb, out_sp, s_sp, semv, sems0, sems1, semi)


def _combine_body(p0_ref, p1_ref, x_ref, ws_ref, bs_ref, o_ref):
    t = jnp.sum(p0_ref[...], axis=0) + jnp.sum(p1_ref[...], axis=0)
    acc = t[:, :D] + t[:, D:]
    o_ref[...] = acc * jnp.float32(1.0 / H) + jnp.dot(
        x_ref[...], ws_ref[...], preferred_element_type=jnp.float32) + bs_ref[...]


def _combine(p0, p1, x, Wskip, bskip):
    grid = (N // BN,)
    return pl.pallas_call(
        _combine_body,
        grid=grid,
        in_specs=[
            pl.BlockSpec((4, BN, 2 * D), lambda i: (0, i, 0)),
            pl.BlockSpec((4, BN, 2 * D), lambda i: (0, i, 0)),
            pl.BlockSpec((BN, F), lambda i: (i, 0)),
            pl.BlockSpec((F, D), lambda i: (0, 0)),
            pl.BlockSpec((1, D), lambda i: (0, 0)),
        ],
        out_specs=pl.BlockSpec((BN, D), lambda i: (i, 0)),
        out_shape=jax.ShapeDtypeStruct((N, D), jnp.float32),
    )(p0, p1, x, Wskip, bskip.reshape(1, D))


def kernel(x, edge_index, Wq, bq, Wk, bk, Wv, bv, Wskip, bskip):
    src = edge_index[0]
    dst = edge_index[1]
    srcp = jnp.concatenate([src, jnp.zeros((EP - E,), jnp.int32)])
    dstp = jnp.concatenate([dst, jnp.zeros((EP - E,), jnp.int32)])
    q, k, vt = _projections(x, Wq, bq, Wk, bk, Wv, bv)

    alpha_flat = _sc_alpha(q, k, srcp, dstp)
    spart = _sc_segsum(alpha_flat, dstp)
    outp = _sc_agg(vt, alpha_flat, srcp, dstp, spart)
    return _combine(outp[0], outp[1], x, Wskip, bskip)
